# Initial kernel scaffold; baseline (speedup 1.0000x reference)
import jax
import jax.numpy as jnp
from jax.experimental import pallas as pl


def kernel(x, edge_index, edge_weight, W1_0, b1_0, W1_1, b1_1, W2_0, b2_0, W2_1, b2_1):
    raise NotImplementedError("write your pallas kernel here")


# speedup vs baseline: 4.7464x; 4.7464x over previous
import functools

import jax
import jax.numpy as jnp
from jax import lax
from jax.experimental import pallas as pl
from jax.experimental.pallas import tpu as pltpu
from jax.experimental.pallas import tpu_sc as plsc

_G = 128


@functools.cache
def _spmm_partials_kernel(n, d, e):
    info = plsc.get_sparse_core_info()
    nc, ns = info.num_cores, info.num_subcores
    nw = nc * ns
    n_chunks = e // _G
    assert n_chunks * _G == e, "edge count must be a multiple of the chunk size"
    base = n_chunks // nw
    extra = n_chunks % nw
    rpt = (n // ns) & ~7
    last_rows = n - rpt * (ns - 1)
    assert rpt % 8 == 0 and last_rows > 0
    mesh = plsc.VectorSubcoreMesh(core_axis_name="c", subcore_axis_name="s")

    @functools.partial(
        pl.kernel,
        out_type=jax.ShapeDtypeStruct((nc, n, d), jnp.float32),
        mesh=mesh,
        scratch_types=[
            pltpu.VMEM_SHARED((n, d), jnp.float32),
            pltpu.VMEM((_G,), jnp.int32),
            pltpu.VMEM((_G,), jnp.int32),
            pltpu.VMEM((_G,), jnp.float32),
            pltpu.VMEM((_G, d), jnp.float32),
            pltpu.SemaphoreType.DMA,
        ],
    )
    def spmm(table_hbm, src_hbm, dst_hbm, w_hbm, zeros_hbm, out_hbm,
             acc, src_buf, dst_buf, w_buf, rows, sem):
        c = lax.axis_index("c")
        s = lax.axis_index("s")
        wid = c * ns + s

        r0 = s * rpt

        @pl.when(s < ns - 1)
        def _():
            pltpu.sync_copy(zeros_hbm.at[pl.ds(r0, rpt)],
                            acc.at[pl.ds(r0, rpt)])

        @pl.when(s == ns - 1)
        def _():
            pltpu.sync_copy(zeros_hbm.at[pl.ds(r0, last_rows)],
                            acc.at[pl.ds(r0, last_rows)])

        plsc.subcore_barrier()

        my_chunks = base + jnp.where(wid < extra, 1, 0)
        start = wid * base + jnp.minimum(wid, extra)

        def chunk_body(i, carry):
            off = (start + i) * _G
            pltpu.sync_copy(src_hbm.at[pl.ds(off, _G)], src_buf)
            pltpu.sync_copy(dst_hbm.at[pl.ds(off, _G)], dst_buf)
            pltpu.sync_copy(w_hbm.at[pl.ds(off, _G)], w_buf)
            pltpu.async_copy(table_hbm.at[src_buf], rows, sem).wait()

            def group_body(g, carry2):
                wvec = w_buf[pl.ds(g * 16, 16)]
                for l in range(16):
                    wv = wvec[l]
                    t = g * 16 + l
                    for j in range(d // 16):
                        sl = pl.ds(j * 16, 16)
                        rows[t, sl] = rows[t, sl] * wv
                return carry2

            lax.fori_loop(0, _G // 16, group_body, 0)
            pltpu.sync_copy(rows, acc.at[dst_buf], add=True)
            return carry

        lax.fori_loop(0, my_chunks, chunk_body, 0)
        plsc.subcore_barrier()

        @pl.when(s < ns - 1)
        def _():
            pltpu.sync_copy(acc.at[pl.ds(r0, rpt)],
                            out_hbm.at[c, pl.ds(r0, rpt)])

        @pl.when(s == ns - 1)
        def _():
            pltpu.sync_copy(acc.at[pl.ds(r0, last_rows)],
                            out_hbm.at[c, pl.ds(r0, last_rows)])

    return spmm


def _tc_stage1(p, wcat, bcat):
    _, n, d = p.shape
    blk = 400
    assert n % blk == 0

    def body(p_ref, w_ref, b_ref, h1_ref, h2_ref):
        h = p_ref[0] + p_ref[1]
        y = jnp.dot(h, w_ref[...], preferred_element_type=jnp.float32)
        y = jnp.maximum(y + b_ref[...], 0.0)
        h1_ref[...] = y[:, :d]
        h2_ref[...] = y[:, d:]

    return pl.pallas_call(
        body,
        grid=(n // blk,),
        in_specs=[
            pl.BlockSpec((2, blk, d), lambda i: (0, i, 0)),
            pl.BlockSpec((d, 2 * d), lambda i: (0, 0)),
            pl.BlockSpec((2 * d,), lambda i: (0,)),
        ],
        out_specs=[
            pl.BlockSpec((blk, d), lambda i: (i, 0)),
            pl.BlockSpec((blk, d), lambda i: (i, 0)),
        ],
        out_shape=[
            jax.ShapeDtypeStruct((n, d), jnp.float32),
            jax.ShapeDtypeStruct((n, d), jnp.float32),
        ],
    )(p, wcat, bcat)


def _tc_stage2(pb, pc, w1, b1, w2, b2):
    _, n, d = pb.shape
    blk = 400
    assert n % blk == 0

    def body(pb_ref, pc_ref, w1_ref, b1_ref, w2_ref, b2_ref, x1_ref, x2_ref):
        s1 = pb_ref[0] + pb_ref[1]
        s2 = pc_ref[0] + pc_ref[1]
        x1_ref[...] = jnp.dot(s1, w1_ref[...],
                              preferred_element_type=jnp.float32) + b1_ref[...]
        x2_ref[...] = jnp.dot(s2, w2_ref[...],
                              preferred_element_type=jnp.float32) + b2_ref[...]

    return pl.pallas_call(
        body,
        grid=(n // blk,),
        in_specs=[
            pl.BlockSpec((2, blk, d), lambda i: (0, i, 0)),
            pl.BlockSpec((2, blk, d), lambda i: (0, i, 0)),
            pl.BlockSpec((d, d), lambda i: (0, 0)),
            pl.BlockSpec((d,), lambda i: (0,)),
            pl.BlockSpec((d, d), lambda i: (0, 0)),
            pl.BlockSpec((d,), lambda i: (0,)),
        ],
        out_specs=[
            pl.BlockSpec((blk, d), lambda i: (i, 0)),
            pl.BlockSpec((blk, d), lambda i: (i, 0)),
        ],
        out_shape=[
            jax.ShapeDtypeStruct((n, d), jnp.float32),
            jax.ShapeDtypeStruct((n, d), jnp.float32),
        ],
    )(pb, pc, w1, b1, w2, b2)


def kernel(x, edge_index, edge_weight, W1_0, b1_0, W1_1, b1_1,
           W2_0, b2_0, W2_1, b2_1):
    n, d = x.shape
    e = edge_weight.shape[0]
    dst = edge_index[0]
    src = edge_index[1]
    zeros = jnp.zeros((n, d), jnp.float32)

    spmm = _spmm_partials_kernel(n, d, e)
    pa = spmm(x, src, dst, edge_weight, zeros)

    wcat = jnp.concatenate([W1_0, W2_0], axis=1)
    bcat = jnp.concatenate([b1_0, b2_0], axis=0)
    h1, h2 = _tc_stage1(pa, wcat, bcat)

    pb = spmm(h1, src, dst, edge_weight, zeros)
    pc = spmm(h2, src, dst, edge_weight, zeros)

    x1, x2 = _tc_stage2(pb, pc, W1_1, b1_1, W2_1, b2_1)
    return (x1, x2)
